# Initial kernel scaffold; baseline (speedup 1.0000x reference)
import jax
import jax.numpy as jnp
from jax.experimental import pallas as pl


def kernel(x, edge_index, batch, W1_0, b1_0, gamma_0, beta_0, W2_0, b2_0, W1_1, b1_1, gamma_1, beta_1, W2_1, b2_1, W1_2, b1_2, gamma_2, beta_2, W2_2, b2_2, Wd1, bd1, Wd2, bd2):
    raise NotImplementedError("write your pallas kernel here")


# speedup vs baseline: 3.1659x; 3.1659x over previous
import functools

import jax
import jax.numpy as jnp
from jax import lax
from jax.experimental import pallas as pl
from jax.experimental.pallas import tpu as pltpu
from jax.experimental.pallas import tpu_sc as plsc

N = 10000
E = 320000
D_IN = 128
H = 256
G = 64
C = 10
BN_EPS = 1e-5

N_PAD = 10240
E_PAD = 327680
E_ROWS = E_PAD // 128
RB = 1024
N_BLK = N_PAD // RB

_f32 = jnp.float32
_i32 = jnp.int32


def _make_sc_agg(split_edges: bool):
  rt = (E_ROWS // 32) if split_edges else (E_ROWS // 16)
  ngrp = rt // 8
  mesh = plsc.VectorSubcoreMesh(core_axis_name="c", subcore_axis_name="s")

  @functools.partial(
      pl.kernel,
      mesh=mesh,
      out_type=[
          jax.ShapeDtypeStruct((N_PAD, 128), _f32),
          jax.ShapeDtypeStruct((N_PAD, 128), _f32),
      ],
      scratch_types=[
          pltpu.VMEM((2, 8, 128), _i32),
          pltpu.VMEM((2, 8, 128), _i32),
          pltpu.VMEM((2, 128, 128), _f32),
          pltpu.VMEM((64, 128), _f32),
          pltpu.VMEM_SHARED((N_PAD, 128), _f32),
          pltpu.SemaphoreType.DMA,
          pltpu.SemaphoreType.DMA,
          pltpu.SemaphoreType.DMA,
          pltpu.SemaphoreType.DMA,
      ],
  )
  def sc_agg(h0, h1, src_r, dst_r, out0, out1,
             src_v, dst_v, rows_v, cp_v, acc,
             semg0, semg1, semi0, semi1):
    c = lax.axis_index("c")
    s = lax.axis_index("s")
    semg = (semg0, semg1)
    semi = (semi0, semi1)

    zero16 = jnp.zeros((16,), _f32)

    @pl.loop(0, 64)
    def _(i):
      for j in range(8):
        cp_v[i, pl.ds(j * 16, 16)] = zero16

    base_out = pl.multiple_of(s * 640, 64)
    for p in range(10):
      pltpu.sync_copy(cp_v, acc.at[pl.ds(base_out + p * 64, 64)])

    if split_edges:
      row0 = pl.multiple_of(c * (E_ROWS // 2) + s * rt, 8)
    else:
      row0 = pl.multiple_of(s * rt, 8)

    def start_idx(g, b):
      pltpu.async_copy(src_r.at[pl.ds(pl.multiple_of(row0 + g * 8, 8), 8)],
                       src_v.at[b], semi[b])
      pltpu.async_copy(dst_r.at[pl.ds(pl.multiple_of(row0 + g * 8, 8), 8)],
                       dst_v.at[b], semi[b])

    def wait_idx(b):
      pltpu.make_async_copy(src_r.at[pl.ds(0, 8)], src_v.at[b], semi[b]).wait()
      pltpu.make_async_copy(dst_r.at[pl.ds(0, 8)], dst_v.at[b], semi[b]).wait()

    def start_gather(gb, k, b):
      @pl.when(c == 0)
      def _():
        pltpu.async_copy(h0.at[src_v.at[gb, k]], rows_v.at[b], semg[b])

      @pl.when(c == 1)
      def _():
        pltpu.async_copy(h1.at[src_v.at[gb, k]], rows_v.at[b], semg[b])

    def wait_gather(b):
      pltpu.make_async_copy(h0.at[src_v.at[0, 0]], rows_v.at[b], semg[b]).wait()

    def scatter(gb, k, b):
      pltpu.sync_copy(rows_v.at[b], acc.at[dst_v.at[gb, k]], add=True)

    start_idx(0, 0)
    wait_idx(0)
    plsc.subcore_barrier()
    start_gather(0, 0, 0)

    @pl.loop(0, ngrp, step=2)
    def _(g):
      for gb in range(2):
        @pl.when(g + gb + 1 < ngrp)
        def _():
          start_idx(g + gb + 1, 1 - gb)

        for k in range(8):
          b = k % 2
          wait_gather(b)
          if k < 7:
            start_gather(gb, k + 1, 1 - b)
            scatter(gb, k, b)
          else:
            @pl.when(g + gb + 1 < ngrp)
            def _():
              wait_idx(1 - gb)
              start_gather(1 - gb, 0, 1 - b)

            scatter(gb, k, b)

    plsc.subcore_barrier()

    for p in range(10):
      sl = pl.ds(base_out + p * 64, 64)
      pltpu.sync_copy(acc.at[sl], cp_v)

      @pl.when(c == 0)
      def _():
        pltpu.sync_copy(cp_v, out0.at[sl])

      @pl.when(c == 1)
      def _():
        pltpu.sync_copy(cp_v, out1.at[sl])

  return sc_agg


def _k1_body(n_in, *refs):
  i = pl.program_id(0)
  ins = refs[:n_in]
  w1_ref = refs[n_in]
  t_ref = refs[n_in + 1]
  stats_ref = refs[n_in + 2]

  if n_in == 3:
    xin = ins[0][...] + ins[1][...] + ins[2][...]
  else:
    xin = jnp.concatenate(
        [ins[0][...] + ins[2][...], ins[1][...] + ins[3][...]], axis=1)

  rid = i * RB + lax.broadcasted_iota(_i32, (RB, 1), 0)
  valid = (rid < N).astype(_f32)
  xin = xin * valid

  t = jnp.dot(xin, w1_ref[...], preferred_element_type=_f32)
  t_ref[...] = t

  tv = t * valid
  psum = jnp.sum(tv, axis=0, keepdims=True)
  psq = jnp.sum(tv * t, axis=0, keepdims=True)
  upd = jnp.concatenate([psum, psq, jnp.zeros((6, H), _f32)], axis=0)

  @pl.when(i == 0)
  def _():
    stats_ref[...] = upd

  @pl.when(i > 0)
  def _():
    stats_ref[...] = stats_ref[...] + upd


def _run_k1(in_parts, w1):
  n_in = len(in_parts)
  d_in = w1.shape[0]
  in_specs = [pl.BlockSpec((RB, 128), lambda i: (i, 0)) for _ in in_parts]
  in_specs.append(pl.BlockSpec((d_in, H), lambda i: (0, 0)))
  return pl.pallas_call(
      functools.partial(_k1_body, n_in),
      grid=(N_BLK,),
      in_specs=in_specs,
      out_specs=[
          pl.BlockSpec((RB, H), lambda i: (i, 0)),
          pl.BlockSpec((8, H), lambda i: (0, 0)),
      ],
      out_shape=[
          jax.ShapeDtypeStruct((N_PAD, H), _f32),
          jax.ShapeDtypeStruct((8, H), _f32),
      ],
  )(*in_parts, w1)


def _k2_body(split_out, t_ref, stats_ref, par_ref, w2_ref, *outs):
  i = pl.program_id(0)
  t = t_ref[...]
  mean = stats_ref[0:1, :] * (1.0 / N)
  ex2 = stats_ref[1:2, :] * (1.0 / N)
  var = ex2 - mean * mean
  inv = lax.rsqrt(var + BN_EPS)
  gamma = par_ref[0:1, :]
  beta = par_ref[1:2, :]
  b2 = par_ref[2:3, :]

  hn = (t - mean) * (inv * gamma) + beta
  hr = jnp.maximum(hn, 0.0)
  out = jnp.dot(hr, w2_ref[...], preferred_element_type=_f32) + b2
  out = jnp.maximum(out, 0.0)

  rid = i * RB + lax.broadcasted_iota(_i32, (RB, 1), 0)
  out = out * (rid < N).astype(_f32)

  if split_out:
    outs[0][...] = out[:, :128]
    outs[1][...] = out[:, 128:]
  else:
    outs[0][...] = out


def _run_k2(t, stats, par, w2, split_out):
  if split_out:
    out_specs = [pl.BlockSpec((RB, 128), lambda i: (i, 0))] * 2
    out_shape = [jax.ShapeDtypeStruct((N_PAD, 128), _f32)] * 2
  else:
    out_specs = [pl.BlockSpec((RB, H), lambda i: (i, 0))]
    out_shape = [jax.ShapeDtypeStruct((N_PAD, H), _f32)]
  return pl.pallas_call(
      functools.partial(_k2_body, split_out),
      grid=(N_BLK,),
      in_specs=[
          pl.BlockSpec((RB, H), lambda i: (i, 0)),
          pl.BlockSpec((8, H), lambda i: (0, 0)),
          pl.BlockSpec((8, H), lambda i: (0, 0)),
          pl.BlockSpec((H, H), lambda i: (0, 0)),
      ],
      out_specs=out_specs,
      out_shape=out_shape,
  )(t, stats, par, w2)


def _pool_body(batch_ref, e0a_ref, e0b_ref, e1a_ref, e1b_ref, e2_ref,
               wd1_ref, bd1_ref, wd2_ref, bd2_ref, z_ref, pooled_ref):
  i = pl.program_id(0)

  @pl.when(i == 0)
  def _():
    pooled_ref[...] = jnp.zeros((G, 3 * H), _f32)

  bb = batch_ref[0, 0, :]
  gids = lax.broadcasted_iota(_i32, (G, RB), 0)
  m = (bb[None, :] == gids).astype(_f32)

  def acc(col, width, ref):
    pooled_ref[:, col:col + width] = pooled_ref[:, col:col + width] + jnp.dot(
        m, ref[...], preferred_element_type=_f32)

  acc(0, 128, e0a_ref)
  acc(128, 128, e0b_ref)
  acc(256, 128, e1a_ref)
  acc(384, 128, e1b_ref)
  acc(512, 256, e2_ref)

  @pl.when(i == N_BLK - 1)
  def _():
    hcat = pooled_ref[...]
    z1 = jnp.dot(hcat, wd1_ref[...], preferred_element_type=_f32)
    z1 = jnp.maximum(z1 + bd1_ref[0:1, :], 0.0)
    z2 = jnp.dot(z1, wd2_ref[...], preferred_element_type=_f32)
    z2 = z2 + bd2_ref[0:1, :]
    col = lax.broadcasted_iota(_i32, (G, 128), 1)
    zm = jnp.where(col < C, z2, -1e30)
    mx = jnp.max(zm, axis=1, keepdims=True)
    lse = jnp.log(jnp.sum(jnp.exp(zm - mx), axis=1, keepdims=True)) + mx
    z_ref[...] = zm - lse


def _run_pool(batch3d, e0a, e0b, e1a, e1b, e2, wd1, bd1r, wd2p, bd2r):
  d_cat = 3 * H
  return pl.pallas_call(
      _pool_body,
      grid=(N_BLK,),
      in_specs=[
          pl.BlockSpec((1, 1, RB), lambda i: (i, 0, 0)),
          pl.BlockSpec((RB, 128), lambda i: (i, 0)),
          pl.BlockSpec((RB, 128), lambda i: (i, 0)),
          pl.BlockSpec((RB, 128), lambda i: (i, 0)),
          pl.BlockSpec((RB, 128), lambda i: (i, 0)),
          pl.BlockSpec((RB, H), lambda i: (i, 0)),
          pl.BlockSpec((d_cat, d_cat), lambda i: (0, 0)),
          pl.BlockSpec((8, d_cat), lambda i: (0, 0)),
          pl.BlockSpec((d_cat, 128), lambda i: (0, 0)),
          pl.BlockSpec((8, 128), lambda i: (0, 0)),
      ],
      out_specs=pl.BlockSpec((G, 128), lambda i: (0, 0)),
      out_shape=jax.ShapeDtypeStruct((G, 128), _f32),
      scratch_shapes=[pltpu.VMEM((G, d_cat), _f32)],
  )(batch3d, e0a, e0b, e1a, e1b, e2, wd1, bd1r, wd2p, bd2r)


def kernel(x, edge_index, batch,
           W1_0, b1_0, gamma_0, beta_0, W2_0, b2_0,
           W1_1, b1_1, gamma_1, beta_1, W2_1, b2_1,
           W1_2, b1_2, gamma_2, beta_2, W2_2, b2_2,
           Wd1, bd1, Wd2, bd2):
  x_pad = jnp.zeros((N_PAD, D_IN), _f32).at[:N].set(x)
  src = jnp.full((E_PAD,), N, _i32).at[:E].set(edge_index[0]).reshape(E_ROWS, 128)
  dst = jnp.full((E_PAD,), N, _i32).at[:E].set(edge_index[1]).reshape(E_ROWS, 128)
  batch3d = jnp.full((N_PAD,), G, _i32).at[:N].set(batch).reshape(N_BLK, 1, RB)

  def pack_par(gamma, beta, b2):
    return jnp.concatenate(
        [gamma[None], beta[None], b2[None], jnp.zeros((5, H), _f32)], axis=0)

  pars = [pack_par(gamma_0, beta_0, b2_0),
          pack_par(gamma_1, beta_1, b2_1),
          pack_par(gamma_2, beta_2, b2_2)]
  w1s = [W1_0, W1_1, W1_2]
  w2s = [W2_0, W2_1, W2_2]

  d_cat = 3 * H
  bd1r = jnp.zeros((8, d_cat), _f32).at[0].set(bd1)
  wd2p = jnp.zeros((d_cat, 128), _f32).at[:, :C].set(Wd2)
  bd2r = jnp.zeros((8, 128), _f32).at[0, :C].set(bd2)

  sc_agg_split = _make_sc_agg(True)
  sc_agg_feat = _make_sc_agg(False)

  aggA, aggB = sc_agg_split(x_pad, x_pad, src, dst)
  t, stats = _run_k1([x_pad, aggA, aggB], w1s[0])
  h0, h1 = _run_k2(t, stats, pars[0], w2s[0], split_out=True)

  agg0, agg1 = sc_agg_feat(h0, h1, src, dst)
  t, stats = _run_k1([h0, h1, agg0, agg1], w1s[1])
  g0, g1 = _run_k2(t, stats, pars[1], w2s[1], split_out=True)

  agg0, agg1 = sc_agg_feat(g0, g1, src, dst)
  t, stats = _run_k1([g0, g1, agg0, agg1], w1s[2])
  (emb2_pad,) = _run_k2(t, stats, pars[2], w2s[2], split_out=False)

  zfull = _run_pool(batch3d, h0, h1, g0, g1, emb2_pad, Wd1, bd1r, wd2p, bd2r)

  return emb2_pad[:N], zfull[:, :C]


# speedup vs baseline: 3.1989x; 1.0104x over previous
import functools

import jax
import jax.numpy as jnp
from jax import lax
from jax.experimental import pallas as pl
from jax.experimental.pallas import tpu as pltpu
from jax.experimental.pallas import tpu_sc as plsc

N = 10000
E = 320000
D_IN = 128
H = 256
G = 64
C = 10
BN_EPS = 1e-5

N_PAD = 10240
E_PAD = 327680
E_ROWS = E_PAD // 128
RB = 1024
N_BLK = N_PAD // RB

_f32 = jnp.float32
_i32 = jnp.int32


def _make_sc_agg(split_edges: bool):
  rt = (E_ROWS // 32) if split_edges else (E_ROWS // 16)
  ngrp = rt // 8
  mesh = plsc.VectorSubcoreMesh(core_axis_name="c", subcore_axis_name="s")

  @functools.partial(
      pl.kernel,
      mesh=mesh,
      out_type=[
          jax.ShapeDtypeStruct((N_PAD, 128), _f32),
          jax.ShapeDtypeStruct((N_PAD, 128), _f32),
      ],
      scratch_types=[
          pltpu.VMEM((2, 8, 128), _i32),
          pltpu.VMEM((2, 8, 128), _i32),
          pltpu.VMEM((2, 128, 128), _f32),
          pltpu.VMEM_SHARED((N_PAD, 128), _f32),
          pltpu.SemaphoreType.DMA,
          pltpu.SemaphoreType.DMA,
          pltpu.SemaphoreType.DMA,
          pltpu.SemaphoreType.DMA,
          pltpu.SemaphoreType.DMA,
          pltpu.SemaphoreType.DMA,
      ],
  )
  def sc_agg(h0, h1, src_r, dst_r, out0, out1,
             src_v, dst_v, rows_v, acc,
             semg0, semg1, semi0, semi1, sems0, sems1):
    c = lax.axis_index("c")
    s = lax.axis_index("s")
    semg = (semg0, semg1)
    semi = (semi0, semi1)
    sems = (sems0, sems1)

    zero16 = jnp.zeros((16,), _f32)

    @pl.loop(0, 128)
    def _(i):
      for bb in range(2):
        for j in range(8):
          rows_v[bb, i, pl.ds(j * 16, 16)] = zero16

    base_out = pl.multiple_of(s * 640, 128)
    for p in range(5):
      pltpu.sync_copy(rows_v.at[p % 2], acc.at[pl.ds(base_out + p * 128, 128)])

    if split_edges:
      row0 = pl.multiple_of(c * (E_ROWS // 2) + s * rt, 8)
    else:
      row0 = pl.multiple_of(s * rt, 8)

    def start_idx(g, b):
      pltpu.async_copy(src_r.at[pl.ds(pl.multiple_of(row0 + g * 8, 8), 8)],
                       src_v.at[b], semi[b])
      pltpu.async_copy(dst_r.at[pl.ds(pl.multiple_of(row0 + g * 8, 8), 8)],
                       dst_v.at[b], semi[b])

    def wait_idx(b):
      pltpu.make_async_copy(src_r.at[pl.ds(0, 8)], src_v.at[b], semi[b]).wait()
      pltpu.make_async_copy(dst_r.at[pl.ds(0, 8)], dst_v.at[b], semi[b]).wait()

    def start_gather(gb, k, b):
      @pl.when(c == 0)
      def _():
        pltpu.async_copy(h0.at[src_v.at[gb, k]], rows_v.at[b], semg[b])

      @pl.when(c == 1)
      def _():
        pltpu.async_copy(h1.at[src_v.at[gb, k]], rows_v.at[b], semg[b])

    def wait_gather(b):
      pltpu.make_async_copy(h0.at[src_v.at[0, 0]], rows_v.at[b], semg[b]).wait()

    def start_scatter(gb, k, b):
      pltpu.make_async_copy(rows_v.at[b], acc.at[dst_v.at[gb, k]],
                            sems[b]).start(add=True)

    def wait_scatter(b):
      pltpu.make_async_copy(rows_v.at[b], acc.at[dst_v.at[0, 0]],
                            sems[b]).wait()

    start_idx(0, 0)
    wait_idx(0)
    plsc.subcore_barrier()
    start_gather(0, 0, 0)

    @pl.loop(0, ngrp, step=2)
    def _(g):
      for gb in range(2):
        @pl.when(g + gb + 1 < ngrp)
        def _():
          start_idx(g + gb + 1, 1 - gb)

        for k in range(8):
          b = k % 2
          wait_gather(b)
          start_scatter(gb, k, b)
          if gb == 0 and k == 0:
            @pl.when(g > 0)
            def _():
              wait_scatter(1 - b)
          else:
            wait_scatter(1 - b)
          if k < 7:
            start_gather(gb, k + 1, 1 - b)
          else:
            @pl.when(g + gb + 1 < ngrp)
            def _():
              wait_idx(1 - gb)
              start_gather(1 - gb, 0, 1 - b)

    wait_scatter(1)
    plsc.subcore_barrier()

    for p in range(5):
      sl = pl.ds(base_out + p * 128, 128)

      @pl.when(c == 0)
      def _():
        pltpu.sync_copy(acc.at[sl], out0.at[sl])

      @pl.when(c == 1)
      def _():
        pltpu.sync_copy(acc.at[sl], out1.at[sl])

  return sc_agg


def _k1_body(n_in, *refs):
  i = pl.program_id(0)
  ins = refs[:n_in]
  w1_ref = refs[n_in]
  t_ref = refs[n_in + 1]
  stats_ref = refs[n_in + 2]

  if n_in == 3:
    xin = ins[0][...] + ins[1][...] + ins[2][...]
  else:
    xin = jnp.concatenate(
        [ins[0][...] + ins[2][...], ins[1][...] + ins[3][...]], axis=1)

  rid = i * RB + lax.broadcasted_iota(_i32, (RB, 1), 0)
  valid = (rid < N).astype(_f32)
  xin = xin * valid

  t = jnp.dot(xin, w1_ref[...], preferred_element_type=_f32)
  t_ref[...] = t

  tv = t * valid
  psum = jnp.sum(tv, axis=0, keepdims=True)
  psq = jnp.sum(tv * t, axis=0, keepdims=True)
  upd = jnp.concatenate([psum, psq, jnp.zeros((6, H), _f32)], axis=0)

  @pl.when(i == 0)
  def _():
    stats_ref[...] = upd

  @pl.when(i > 0)
  def _():
    stats_ref[...] = stats_ref[...] + upd


def _run_k1(in_parts, w1):
  n_in = len(in_parts)
  d_in = w1.shape[0]
  in_specs = [pl.BlockSpec((RB, 128), lambda i: (i, 0)) for _ in in_parts]
  in_specs.append(pl.BlockSpec((d_in, H), lambda i: (0, 0)))
  return pl.pallas_call(
      functools.partial(_k1_body, n_in),
      grid=(N_BLK,),
      in_specs=in_specs,
      out_specs=[
          pl.BlockSpec((RB, H), lambda i: (i, 0)),
          pl.BlockSpec((8, H), lambda i: (0, 0)),
      ],
      out_shape=[
          jax.ShapeDtypeStruct((N_PAD, H), _f32),
          jax.ShapeDtypeStruct((8, H), _f32),
      ],
  )(*in_parts, w1)


def _k2_body(split_out, t_ref, stats_ref, par_ref, w2_ref, *outs):
  i = pl.program_id(0)
  t = t_ref[...]
  mean = stats_ref[0:1, :] * (1.0 / N)
  ex2 = stats_ref[1:2, :] * (1.0 / N)
  var = ex2 - mean * mean
  inv = lax.rsqrt(var + BN_EPS)
  gamma = par_ref[0:1, :]
  beta = par_ref[1:2, :]
  b2 = par_ref[2:3, :]

  hn = (t - mean) * (inv * gamma) + beta
  hr = jnp.maximum(hn, 0.0)
  out = jnp.dot(hr, w2_ref[...], preferred_element_type=_f32) + b2
  out = jnp.maximum(out, 0.0)

  rid = i * RB + lax.broadcasted_iota(_i32, (RB, 1), 0)
  out = out * (rid < N).astype(_f32)

  if split_out:
    outs[0][...] = out[:, :128]
    outs[1][...] = out[:, 128:]
  else:
    outs[0][...] = out


def _run_k2(t, stats, par, w2, split_out):
  if split_out:
    out_specs = [pl.BlockSpec((RB, 128), lambda i: (i, 0))] * 2
    out_shape = [jax.ShapeDtypeStruct((N_PAD, 128), _f32)] * 2
  else:
    out_specs = [pl.BlockSpec((RB, H), lambda i: (i, 0))]
    out_shape = [jax.ShapeDtypeStruct((N_PAD, H), _f32)]
  return pl.pallas_call(
      functools.partial(_k2_body, split_out),
      grid=(N_BLK,),
      in_specs=[
          pl.BlockSpec((RB, H), lambda i: (i, 0)),
          pl.BlockSpec((8, H), lambda i: (0, 0)),
          pl.BlockSpec((8, H), lambda i: (0, 0)),
          pl.BlockSpec((H, H), lambda i: (0, 0)),
      ],
      out_specs=out_specs,
      out_shape=out_shape,
  )(t, stats, par, w2)


def _pool_body(batch_ref, e0a_ref, e0b_ref, e1a_ref, e1b_ref, e2_ref,
               wd1_ref, bd1_ref, wd2_ref, bd2_ref, z_ref, pooled_ref):
  i = pl.program_id(0)

  @pl.when(i == 0)
  def _():
    pooled_ref[...] = jnp.zeros((G, 3 * H), _f32)

  bb = batch_ref[0, 0, :]
  gids = lax.broadcasted_iota(_i32, (G, RB), 0)
  m = (bb[None, :] == gids).astype(_f32)

  def acc(col, width, ref):
    pooled_ref[:, col:col + width] = pooled_ref[:, col:col + width] + jnp.dot(
        m, ref[...], preferred_element_type=_f32)

  acc(0, 128, e0a_ref)
  acc(128, 128, e0b_ref)
  acc(256, 128, e1a_ref)
  acc(384, 128, e1b_ref)
  acc(512, 256, e2_ref)

  @pl.when(i == N_BLK - 1)
  def _():
    hcat = pooled_ref[...]
    z1 = jnp.dot(hcat, wd1_ref[...], preferred_element_type=_f32)
    z1 = jnp.maximum(z1 + bd1_ref[0:1, :], 0.0)
    z2 = jnp.dot(z1, wd2_ref[...], preferred_element_type=_f32)
    z2 = z2 + bd2_ref[0:1, :]
    col = lax.broadcasted_iota(_i32, (G, 128), 1)
    zm = jnp.where(col < C, z2, -1e30)
    mx = jnp.max(zm, axis=1, keepdims=True)
    lse = jnp.log(jnp.sum(jnp.exp(zm - mx), axis=1, keepdims=True)) + mx
    z_ref[...] = zm - lse


def _run_pool(batch3d, e0a, e0b, e1a, e1b, e2, wd1, bd1r, wd2p, bd2r):
  d_cat = 3 * H
  return pl.pallas_call(
      _pool_body,
      grid=(N_BLK,),
      in_specs=[
          pl.BlockSpec((1, 1, RB), lambda i: (i, 0, 0)),
          pl.BlockSpec((RB, 128), lambda i: (i, 0)),
          pl.BlockSpec((RB, 128), lambda i: (i, 0)),
          pl.BlockSpec((RB, 128), lambda i: (i, 0)),
          pl.BlockSpec((RB, 128), lambda i: (i, 0)),
          pl.BlockSpec((RB, H), lambda i: (i, 0)),
          pl.BlockSpec((d_cat, d_cat), lambda i: (0, 0)),
          pl.BlockSpec((8, d_cat), lambda i: (0, 0)),
          pl.BlockSpec((d_cat, 128), lambda i: (0, 0)),
          pl.BlockSpec((8, 128), lambda i: (0, 0)),
      ],
      out_specs=pl.BlockSpec((G, 128), lambda i: (0, 0)),
      out_shape=jax.ShapeDtypeStruct((G, 128), _f32),
      scratch_shapes=[pltpu.VMEM((G, d_cat), _f32)],
  )(batch3d, e0a, e0b, e1a, e1b, e2, wd1, bd1r, wd2p, bd2r)


def kernel(x, edge_index, batch,
           W1_0, b1_0, gamma_0, beta_0, W2_0, b2_0,
           W1_1, b1_1, gamma_1, beta_1, W2_1, b2_1,
           W1_2, b1_2, gamma_2, beta_2, W2_2, b2_2,
           Wd1, bd1, Wd2, bd2):
  x_pad = jnp.zeros((N_PAD, D_IN), _f32).at[:N].set(x)
  src = jnp.full((E_PAD,), N, _i32).at[:E].set(edge_index[0]).reshape(E_ROWS, 128)
  dst = jnp.full((E_PAD,), N, _i32).at[:E].set(edge_index[1]).reshape(E_ROWS, 128)
  batch3d = jnp.full((N_PAD,), G, _i32).at[:N].set(batch).reshape(N_BLK, 1, RB)

  def pack_par(gamma, beta, b2):
    return jnp.concatenate(
        [gamma[None], beta[None], b2[None], jnp.zeros((5, H), _f32)], axis=0)

  pars = [pack_par(gamma_0, beta_0, b2_0),
          pack_par(gamma_1, beta_1, b2_1),
          pack_par(gamma_2, beta_2, b2_2)]
  w1s = [W1_0, W1_1, W1_2]
  w2s = [W2_0, W2_1, W2_2]

  d_cat = 3 * H
  bd1r = jnp.zeros((8, d_cat), _f32).at[0].set(bd1)
  wd2p = jnp.zeros((d_cat, 128), _f32).at[:, :C].set(Wd2)
  bd2r = jnp.zeros((8, 128), _f32).at[0, :C].set(bd2)

  sc_agg_split = _make_sc_agg(True)
  sc_agg_feat = _make_sc_agg(False)

  aggA, aggB = sc_agg_split(x_pad, x_pad, src, dst)
  t, stats = _run_k1([x_pad, aggA, aggB], w1s[0])
  h0, h1 = _run_k2(t, stats, pars[0], w2s[0], split_out=True)

  agg0, agg1 = sc_agg_feat(h0, h1, src, dst)
  t, stats = _run_k1([h0, h1, agg0, agg1], w1s[1])
  g0, g1 = _run_k2(t, stats, pars[1], w2s[1], split_out=True)

  agg0, agg1 = sc_agg_feat(g0, g1, src, dst)
  t, stats = _run_k1([g0, g1, agg0, agg1], w1s[2])
  (emb2_pad,) = _run_k2(t, stats, pars[2], w2s[2], split_out=False)

  zfull = _run_pool(batch3d, h0, h1, g0, g1, emb2_pad, Wd1, bd1r, wd2p, bd2r)

  return emb2_pad[:N], zfull[:, :C]
